# Initial kernel scaffold; baseline (speedup 1.0000x reference)
import jax
import jax.numpy as jnp
from jax.experimental import pallas as pl


def kernel(x, edge_index, W1, a_src1, a_dst1, b1, W2, a_src2, a_dst2, b2):
    raise NotImplementedError("write your pallas kernel here")


# speedup vs baseline: 15.0260x; 15.0260x over previous
import dataclasses
import functools

import jax
import jax.numpy as jnp
from jax import lax
from jax.experimental import pallas as pl
from jax.experimental.pallas import tpu as pltpu
from jax.experimental.pallas import tpu_sc as plsc

N = 10000
D = 128
NC = 2
NS = 16
NW = NC * NS
C = 128
NPAD = 10112
RPT = NPAD // NS

f32 = jnp.float32
i32 = jnp.int32


def _tc_in_proj(x, W, A):
    n = x.shape[0]

    def body(x_ref, w_ref, a_ref, h_ref, avs_ref, avd_ref):
        h = jnp.dot(x_ref[...], w_ref[...], preferred_element_type=f32,
                    precision=lax.Precision.HIGHEST)
        h_ref[...] = h
        av = jnp.dot(h, a_ref[...], preferred_element_type=f32,
                     precision=lax.Precision.HIGHEST)
        avs_ref[...] = jnp.broadcast_to(av[:, 0:1], (n, 16))
        avd_ref[...] = jnp.broadcast_to(av[:, 1:2], (n, 16))

    return pl.pallas_call(
        body,
        out_shape=(jax.ShapeDtypeStruct((n, D), f32),
                   jax.ShapeDtypeStruct((n, 16), f32),
                   jax.ShapeDtypeStruct((n, 16), f32)),
    )(x, W, A)


def _tc_combine_proj(num, den, b, W, A):
    def body(n_ref, d_ref, b_ref, w_ref, a_ref, h_ref, avs_ref, avd_ref):
        nm = n_ref[0] + n_ref[1]
        dn = d_ref[0, :, 0:1] + d_ref[1, :, 0:1]
        hin = jnp.maximum(nm / (dn + 1e-16) + b_ref[...], 0.0)
        h = jnp.dot(hin, w_ref[...], preferred_element_type=f32,
                    precision=lax.Precision.HIGHEST)
        h_ref[...] = h
        av = jnp.dot(h, a_ref[...], preferred_element_type=f32,
                     precision=lax.Precision.HIGHEST)
        avs_ref[...] = jnp.broadcast_to(av[:, 0:1], (NPAD, 16))
        avd_ref[...] = jnp.broadcast_to(av[:, 1:2], (NPAD, 16))

    return pl.pallas_call(
        body,
        out_shape=(jax.ShapeDtypeStruct((NPAD, D), f32),
                   jax.ShapeDtypeStruct((NPAD, 16), f32),
                   jax.ShapeDtypeStruct((NPAD, 16), f32)),
    )(num, den, b.reshape(1, D), W, A)


def _tc_combine_final(num, den, b):
    def body(n_ref, d_ref, b_ref, o_ref):
        nm = n_ref[0] + n_ref[1]
        dn = d_ref[0, :, 0:1] + d_ref[1, :, 0:1]
        o_ref[...] = nm / (dn + 1e-16) + b_ref[...]

    return pl.pallas_call(
        body,
        out_shape=jax.ShapeDtypeStruct((NPAD, D), f32),
    )(num, den, b.reshape(1, D))


def _sc_gat_scatter(h, avs, avd, sidx, didx, z128, z16):
    NCH = sidx.shape[1]
    mesh = plsc.VectorSubcoreMesh(core_axis_name="c", subcore_axis_name="s",
                                  num_cores=NC, num_subcores=NS)
    cp = pltpu.CompilerParams(use_tc_tiling_on_sc=False)
    if "needs_layout_passes" in pltpu.CompilerParams.__dataclass_fields__:
        cp = dataclasses.replace(cp, needs_layout_passes=False)

    @functools.partial(
        pl.kernel,
        compiler_params=cp,
        out_type=(jax.ShapeDtypeStruct((NC, NPAD, D), f32),
                  jax.ShapeDtypeStruct((NC, NPAD, 16), f32)),
        mesh=mesh,
        scratch_types=[
            pltpu.VMEM((1, C), i32),
            pltpu.VMEM((1, C), i32),
            pltpu.VMEM((C, 16), f32),
            pltpu.VMEM((C, 16), f32),
            pltpu.VMEM((C, D), f32),
            pltpu.VMEM((C, 16), f32),
            pltpu.VMEM_SHARED((NPAD, D), f32),
            pltpu.VMEM_SHARED((NPAD, 16), f32),
        ],
    )
    def k(h_hbm, avs_hbm, avd_hbm, s_hbm, d_hbm, z128_hbm, z16_hbm,
          num_out, den_out,
          si_v, di_v, as_v, ad_v, g_v, dn_v, acc_n, acc_d):
        cid = lax.axis_index("c")
        sid = lax.axis_index("s")
        wid = cid * NS + sid
        r0 = sid * RPT

        pltpu.sync_copy(z128_hbm.at[sid], acc_n.at[pl.ds(r0, RPT)])
        pltpu.sync_copy(z16_hbm.at[sid], acc_d.at[pl.ds(r0, RPT)])
        plsc.subcore_barrier()

        @pl.loop(0, NCH)
        def _(j):
            pltpu.sync_copy(s_hbm.at[wid, j], si_v)
            pltpu.sync_copy(d_hbm.at[wid, j], di_v)
            si = si_v.at[0]
            di = di_v.at[0]
            pltpu.sync_copy(avs_hbm.at[si], as_v)
            pltpu.sync_copy(avd_hbm.at[di], ad_v)
            pltpu.sync_copy(h_hbm.at[si], g_v)

            @pl.loop(0, C)
            def _(b):
                s = as_v[b, pl.ds(0, 16)] + ad_v[b, pl.ds(0, 16)]
                w = jnp.exp(jnp.maximum(s, 0.2 * s))
                for m in range(D // 16):
                    g_v[b, pl.ds(m * 16, 16)] = g_v[b, pl.ds(m * 16, 16)] * w
                dn_v[b, pl.ds(0, 16)] = w

            pltpu.sync_copy(g_v, acc_n.at[di], add=True)
            pltpu.sync_copy(dn_v, acc_d.at[di], add=True)

        plsc.subcore_barrier()

        pltpu.sync_copy(acc_n.at[pl.ds(r0, RPT)],
                        num_out.at[cid, pl.ds(r0, RPT)])
        pltpu.sync_copy(acc_d.at[pl.ds(r0, RPT)],
                        den_out.at[cid, pl.ds(r0, RPT)])

    return k(h, avs, avd, sidx, didx, z128, z16)


def kernel(x, edge_index, W1, a_src1, a_dst1, b1, W2, a_src2, a_dst2, b2):
    E = edge_index.shape[1]
    ET = E + N
    NCH = -(-ET // (NW * C))
    pad = NW * NCH * C - ET

    loop_idx = jnp.arange(N, dtype=i32)
    src = jnp.concatenate(
        [edge_index[0], loop_idx,
         jnp.zeros((pad,), i32)]).reshape(NW, NCH, 1, C)
    dst = jnp.concatenate(
        [edge_index[1], loop_idx,
         jnp.full((pad,), N, i32)]).reshape(NW, NCH, 1, C)

    A1 = jnp.stack([a_src1, a_dst1], axis=1)
    A2 = jnp.stack([a_src2, a_dst2], axis=1)
    z128 = jnp.zeros((NS, RPT, D), f32)
    z16 = jnp.zeros((NS, RPT, 16), f32)

    h1, avs1, avd1 = _tc_in_proj(x, W1, A1)
    avs1p = jnp.pad(avs1, ((0, NPAD - N), (0, 0)))
    avd1p = jnp.pad(avd1, ((0, NPAD - N), (0, 0)))
    n1, d1 = _sc_gat_scatter(h1, avs1p, avd1p, src, dst, z128, z16)
    h2, avs2, avd2 = _tc_combine_proj(n1, d1, b1, W2, A2)
    n2, d2 = _sc_gat_scatter(h2, avs2, avd2, src, dst, z128, z16)
    out = _tc_combine_final(n2, d2, b2)
    return out[:N]


# speedup vs baseline: 20.0955x; 1.3374x over previous
import dataclasses
import functools

import jax
import jax.numpy as jnp
from jax import lax
from jax.experimental import pallas as pl
from jax.experimental.pallas import tpu as pltpu
from jax.experimental.pallas import tpu_sc as plsc

N = 10000
D = 128
NC = 2
NS = 16
NW = NC * NS
C = 128
NPAD = 10112
RPT = NPAD // NS

f32 = jnp.float32
i32 = jnp.int32


def _tc_in_proj(x, W, A):
    n = x.shape[0]

    def body(x_ref, w_ref, a_ref, h_ref, avs_ref, avd_ref):
        h = jnp.dot(x_ref[...], w_ref[...], preferred_element_type=f32,
                    precision=lax.Precision.HIGHEST)
        h_ref[...] = h
        av = jnp.dot(h, a_ref[...], preferred_element_type=f32,
                     precision=lax.Precision.HIGHEST)
        avs_ref[...] = jnp.broadcast_to(av[:, 0:1], (n, 16))
        avd_ref[...] = jnp.broadcast_to(av[:, 1:2], (n, 16))

    return pl.pallas_call(
        body,
        out_shape=(jax.ShapeDtypeStruct((n, D), f32),
                   jax.ShapeDtypeStruct((n, 16), f32),
                   jax.ShapeDtypeStruct((n, 16), f32)),
    )(x, W, A)


def _tc_combine_proj(num, den, b, W, A):
    def body(n_ref, d_ref, b_ref, w_ref, a_ref, h_ref, avs_ref, avd_ref):
        nm = n_ref[0] + n_ref[1]
        dn = d_ref[0, :, 0:1] + d_ref[1, :, 0:1]
        hin = jnp.maximum(nm / (dn + 1e-16) + b_ref[...], 0.0)
        h = jnp.dot(hin, w_ref[...], preferred_element_type=f32,
                    precision=lax.Precision.HIGHEST)
        h_ref[...] = h
        av = jnp.dot(h, a_ref[...], preferred_element_type=f32,
                     precision=lax.Precision.HIGHEST)
        avs_ref[...] = jnp.broadcast_to(av[:, 0:1], (NPAD, 16))
        avd_ref[...] = jnp.broadcast_to(av[:, 1:2], (NPAD, 16))

    return pl.pallas_call(
        body,
        out_shape=(jax.ShapeDtypeStruct((NPAD, D), f32),
                   jax.ShapeDtypeStruct((NPAD, 16), f32),
                   jax.ShapeDtypeStruct((NPAD, 16), f32)),
    )(num, den, b.reshape(1, D), W, A)


def _tc_combine_final(num, den, b):
    def body(n_ref, d_ref, b_ref, o_ref):
        nm = n_ref[0] + n_ref[1]
        dn = d_ref[0, :, 0:1] + d_ref[1, :, 0:1]
        o_ref[...] = nm / (dn + 1e-16) + b_ref[...]

    return pl.pallas_call(
        body,
        out_shape=jax.ShapeDtypeStruct((NPAD, D), f32),
    )(num, den, b.reshape(1, D))


def _sc_gat_scatter(h, avs, avd, sidx, didx, z128, z16):
    NCH = sidx.shape[1]
    mesh = plsc.VectorSubcoreMesh(core_axis_name="c", subcore_axis_name="s",
                                  num_cores=NC, num_subcores=NS)
    cp = pltpu.CompilerParams(use_tc_tiling_on_sc=False)
    if "needs_layout_passes" in pltpu.CompilerParams.__dataclass_fields__:
        cp = dataclasses.replace(cp, needs_layout_passes=False)

    @functools.partial(
        pl.kernel,
        compiler_params=cp,
        out_type=(jax.ShapeDtypeStruct((NC, NPAD, D), f32),
                  jax.ShapeDtypeStruct((NC, NPAD, 16), f32)),
        mesh=mesh,
        scratch_types=[
            pltpu.VMEM((1, C), i32),
            pltpu.VMEM((1, C), i32),
            pltpu.VMEM((C, 16), f32),
            pltpu.VMEM((C, 16), f32),
            pltpu.VMEM((C, D), f32),
            pltpu.VMEM((C, 16), f32),
            pltpu.VMEM_SHARED((NPAD, D), f32),
            pltpu.VMEM_SHARED((NPAD, 16), f32),
            pltpu.SemaphoreType.DMA,
            pltpu.SemaphoreType.DMA,
            pltpu.SemaphoreType.DMA,
            pltpu.SemaphoreType.DMA,
            pltpu.SemaphoreType.DMA,
        ],
    )
    def k(h_hbm, avs_hbm, avd_hbm, s_hbm, d_hbm, z128_hbm, z16_hbm,
          num_out, den_out,
          si_v, di_v, as_v, ad_v, g_v, dn_v, acc_n, acc_d,
          sem_g, sem_a, sem_b, sem_s1, sem_s2):
        cid = lax.axis_index("c")
        sid = lax.axis_index("s")
        wid = cid * NS + sid
        r0 = sid * RPT

        pltpu.sync_copy(z128_hbm.at[sid], acc_n.at[pl.ds(r0, RPT)])
        pltpu.sync_copy(z16_hbm.at[sid], acc_d.at[pl.ds(r0, RPT)])
        plsc.subcore_barrier()

        @pl.loop(0, NCH)
        def _(j):
            pltpu.sync_copy(s_hbm.at[wid, j], si_v)
            pltpu.sync_copy(d_hbm.at[wid, j], di_v)
            si = si_v.at[0]
            di = di_v.at[0]
            cg = pltpu.async_copy(h_hbm.at[si], g_v, sem_g)
            ca = pltpu.async_copy(avs_hbm.at[si], as_v, sem_a)
            cb = pltpu.async_copy(avd_hbm.at[di], ad_v, sem_b)
            ca.wait()
            cb.wait()

            @pl.loop(0, C)
            def _(b):
                s = as_v[b, pl.ds(0, 16)] + ad_v[b, pl.ds(0, 16)]
                dn_v[b, pl.ds(0, 16)] = jnp.exp(jnp.maximum(s, 0.2 * s))

            cg.wait()

            @pl.loop(0, C)
            def _(b):
                w = dn_v[b, pl.ds(0, 16)]
                for m in range(D // 16):
                    g_v[b, pl.ds(m * 16, 16)] = g_v[b, pl.ds(m * 16, 16)] * w

            cs1 = pltpu.async_copy(g_v, acc_n.at[di], sem_s1, add=True)
            cs2 = pltpu.async_copy(dn_v, acc_d.at[di], sem_s2, add=True)
            cs1.wait()
            cs2.wait()

        plsc.subcore_barrier()

        pltpu.sync_copy(acc_n.at[pl.ds(r0, RPT)],
                        num_out.at[cid, pl.ds(r0, RPT)])
        pltpu.sync_copy(acc_d.at[pl.ds(r0, RPT)],
                        den_out.at[cid, pl.ds(r0, RPT)])

    return k(h, avs, avd, sidx, didx, z128, z16)


def kernel(x, edge_index, W1, a_src1, a_dst1, b1, W2, a_src2, a_dst2, b2):
    E = edge_index.shape[1]
    ET = E + N
    NCH = -(-ET // (NW * C))
    pad = NW * NCH * C - ET

    loop_idx = jnp.arange(N, dtype=i32)
    src = jnp.concatenate(
        [edge_index[0], loop_idx,
         jnp.zeros((pad,), i32)]).reshape(NW, NCH, 1, C)
    dst = jnp.concatenate(
        [edge_index[1], loop_idx,
         jnp.full((pad,), N, i32)]).reshape(NW, NCH, 1, C)

    A1 = jnp.stack([a_src1, a_dst1], axis=1)
    A2 = jnp.stack([a_src2, a_dst2], axis=1)
    z128 = jnp.zeros((NS, RPT, D), f32)
    z16 = jnp.zeros((NS, RPT, 16), f32)

    h1, avs1, avd1 = _tc_in_proj(x, W1, A1)
    avs1p = jnp.pad(avs1, ((0, NPAD - N), (0, 0)))
    avd1p = jnp.pad(avd1, ((0, NPAD - N), (0, 0)))
    n1, d1 = _sc_gat_scatter(h1, avs1p, avd1p, src, dst, z128, z16)
    h2, avs2, avd2 = _tc_combine_proj(n1, d1, b1, W2, A2)
    n2, d2 = _sc_gat_scatter(h2, avs2, avd2, src, dst, z128, z16)
    out = _tc_combine_final(n2, d2, b2)
    return out[:N]


# speedup vs baseline: 26.9305x; 1.3401x over previous
import dataclasses
import functools

import jax
import jax.numpy as jnp
from jax import lax
from jax.experimental import pallas as pl
from jax.experimental.pallas import tpu as pltpu
from jax.experimental.pallas import tpu_sc as plsc

N = 10000
D = 128
NC = 2
NS = 16
NW = NC * NS
C = 120
NPAD = 10016
RPT = NPAD // NS

f32 = jnp.float32
i32 = jnp.int32


def _tc_in_proj(x, W, A):
    n = x.shape[0]

    def body(x_ref, w_ref, a_ref, h_ref, avs_ref, avd_ref):
        h = jnp.dot(x_ref[...], w_ref[...], preferred_element_type=f32,
                    precision=lax.Precision.HIGHEST)
        h_ref[...] = h
        av = jnp.dot(h, a_ref[...], preferred_element_type=f32,
                     precision=lax.Precision.HIGHEST)
        avs_ref[...] = jnp.broadcast_to(av[:, 0:1], (n, 16))
        avd_ref[...] = jnp.broadcast_to(av[:, 1:2], (n, 16))

    return pl.pallas_call(
        body,
        out_shape=(jax.ShapeDtypeStruct((n, D), f32),
                   jax.ShapeDtypeStruct((n, 16), f32),
                   jax.ShapeDtypeStruct((n, 16), f32)),
    )(x, W, A)


def _tc_combine_proj(num, den, b, W, A):
    def body(n_ref, d_ref, b_ref, w_ref, a_ref, h_ref, avs_ref, avd_ref):
        nm = n_ref[0] + n_ref[1]
        dn = d_ref[0, :, 0:1] + d_ref[1, :, 0:1]
        hin = jnp.maximum(nm / (dn + 1e-16) + b_ref[...], 0.0)
        h = jnp.dot(hin, w_ref[...], preferred_element_type=f32,
                    precision=lax.Precision.HIGHEST)
        h_ref[...] = h
        av = jnp.dot(h, a_ref[...], preferred_element_type=f32,
                     precision=lax.Precision.HIGHEST)
        avs_ref[...] = jnp.broadcast_to(av[:, 0:1], (NPAD, 16))
        avd_ref[...] = jnp.broadcast_to(av[:, 1:2], (NPAD, 16))

    return pl.pallas_call(
        body,
        out_shape=(jax.ShapeDtypeStruct((NPAD, D), f32),
                   jax.ShapeDtypeStruct((NPAD, 16), f32),
                   jax.ShapeDtypeStruct((NPAD, 16), f32)),
    )(num, den, b.reshape(1, D), W, A)


def _tc_combine_final(num, den, b):
    def body(n_ref, d_ref, b_ref, o_ref):
        nm = n_ref[0] + n_ref[1]
        dn = d_ref[0, :, 0:1] + d_ref[1, :, 0:1]
        o_ref[...] = nm / (dn + 1e-16) + b_ref[...]

    return pl.pallas_call(
        body,
        out_shape=jax.ShapeDtypeStruct((NPAD, D), f32),
    )(num, den, b.reshape(1, D))


def _sc_gat_scatter(h, avs, avd, sidx, didx, z128, z16):
    NCH = sidx.shape[1]
    mesh = plsc.VectorSubcoreMesh(core_axis_name="c", subcore_axis_name="s",
                                  num_cores=NC, num_subcores=NS)
    cp = pltpu.CompilerParams(use_tc_tiling_on_sc=False)
    if "needs_layout_passes" in pltpu.CompilerParams.__dataclass_fields__:
        cp = dataclasses.replace(cp, needs_layout_passes=False)

    @functools.partial(
        pl.kernel,
        compiler_params=cp,
        out_type=(jax.ShapeDtypeStruct((NC, NPAD, D), f32),
                  jax.ShapeDtypeStruct((NC, NPAD, 16), f32)),
        mesh=mesh,
        scratch_types=[
            pltpu.VMEM((1, C), i32),
            pltpu.VMEM((1, C), i32),
            pltpu.VMEM((1, C), i32),
            pltpu.VMEM((1, C), i32),
            pltpu.VMEM((C, 16), f32),
            pltpu.VMEM((C, 16), f32),
            pltpu.VMEM((C, D), f32),
            pltpu.VMEM((C, D), f32),
            pltpu.VMEM((C, 16), f32),
            pltpu.VMEM((C, 16), f32),
            pltpu.VMEM_SHARED((NPAD, D), f32),
            pltpu.VMEM_SHARED((NPAD, 16), f32),
            pltpu.SemaphoreType.DMA,
            pltpu.SemaphoreType.DMA,
            pltpu.SemaphoreType.DMA,
            pltpu.SemaphoreType.DMA,
            pltpu.SemaphoreType.DMA,
            pltpu.SemaphoreType.DMA,
            pltpu.SemaphoreType.DMA,
            pltpu.SemaphoreType.DMA,
        ],
    )
    def k(h_hbm, avs_hbm, avd_hbm, s_hbm, d_hbm, z128_hbm, z16_hbm,
          num_out, den_out,
          si0, si1, di0, di1, as_v, ad_v, g0, g1, dn0, dn1, acc_n, acc_d,
          sem_g0, sem_g1, sem_a, sem_b, sem_n0, sem_n1, sem_d0, sem_d1):
        cid = lax.axis_index("c")
        sid = lax.axis_index("s")
        wid = cid * NS + sid
        r0 = sid * RPT

        pltpu.sync_copy(z128_hbm.at[sid], acc_n.at[pl.ds(r0, RPT)])
        pltpu.sync_copy(z16_hbm.at[sid], acc_d.at[pl.ds(r0, RPT)])
        plsc.subcore_barrier()

        drain_n = num_out.at[0, pl.ds(0, C)]
        drain_d = den_out.at[0, pl.ds(0, C)]

        def half(j, si_p, di_p, g_p, dn_p, sem_g_p, sem_n_p, sem_d_p):
            @pl.when(j >= 2)
            def _():
                pltpu.make_async_copy(drain_n, g_p, sem_n_p).wait()
                pltpu.make_async_copy(drain_d, dn_p, sem_d_p).wait()

            pltpu.sync_copy(s_hbm.at[wid, j], si_p)
            pltpu.sync_copy(d_hbm.at[wid, j], di_p)
            si = si_p.at[0]
            di = di_p.at[0]
            cg = pltpu.async_copy(h_hbm.at[si], g_p, sem_g_p)
            ca = pltpu.async_copy(avs_hbm.at[si], as_v, sem_a)
            cb = pltpu.async_copy(avd_hbm.at[di], ad_v, sem_b)
            ca.wait()
            cb.wait()

            @pl.loop(0, C)
            def _(b):
                s = as_v[b, pl.ds(0, 16)] + ad_v[b, pl.ds(0, 16)]
                dn_p[b, pl.ds(0, 16)] = jnp.exp(jnp.maximum(s, 0.2 * s))

            cg.wait()

            @pl.loop(0, C)
            def _(b):
                w = dn_p[b, pl.ds(0, 16)]
                for m in range(D // 16):
                    g_p[b, pl.ds(m * 16, 16)] = g_p[b, pl.ds(m * 16, 16)] * w

            pltpu.async_copy(g_p, acc_n.at[di], sem_n_p, add=True)
            pltpu.async_copy(dn_p, acc_d.at[di], sem_d_p, add=True)

        @pl.loop(0, NCH, step=2)
        def _(jj):
            half(jj, si0, di0, g0, dn0, sem_g0, sem_n0, sem_d0)
            half(jj + 1, si1, di1, g1, dn1, sem_g1, sem_n1, sem_d1)

        pltpu.make_async_copy(drain_n, g0, sem_n0).wait()
        pltpu.make_async_copy(drain_d, dn0, sem_d0).wait()
        pltpu.make_async_copy(drain_n, g1, sem_n1).wait()
        pltpu.make_async_copy(drain_d, dn1, sem_d1).wait()

        plsc.subcore_barrier()

        pltpu.sync_copy(acc_n.at[pl.ds(r0, RPT)],
                        num_out.at[cid, pl.ds(r0, RPT)])
        pltpu.sync_copy(acc_d.at[pl.ds(r0, RPT)],
                        den_out.at[cid, pl.ds(r0, RPT)])

    return k(h, avs, avd, sidx, didx, z128, z16)


def kernel(x, edge_index, W1, a_src1, a_dst1, b1, W2, a_src2, a_dst2, b2):
    E = edge_index.shape[1]
    ET = E + N
    NCH = -(-ET // (NW * C))
    NCH += NCH % 2
    pad = NW * NCH * C - ET

    loop_idx = jnp.arange(N, dtype=i32)
    src = jnp.concatenate(
        [edge_index[0], loop_idx,
         jnp.zeros((pad,), i32)]).reshape(NW, NCH, 1, C)
    dst = jnp.concatenate(
        [edge_index[1], loop_idx,
         jnp.full((pad,), N, i32)]).reshape(NW, NCH, 1, C)

    A1 = jnp.stack([a_src1, a_dst1], axis=1)
    A2 = jnp.stack([a_src2, a_dst2], axis=1)
    z128 = jnp.zeros((NS, RPT, D), f32)
    z16 = jnp.zeros((NS, RPT, 16), f32)

    h1, avs1, avd1 = _tc_in_proj(x, W1, A1)
    avs1p = jnp.pad(avs1, ((0, NPAD - N), (0, 0)))
    avd1p = jnp.pad(avd1, ((0, NPAD - N), (0, 0)))
    n1, d1 = _sc_gat_scatter(h1, avs1p, avd1p, src, dst, z128, z16)
    h2, avs2, avd2 = _tc_combine_proj(n1, d1, b1, W2, A2)
    n2, d2 = _sc_gat_scatter(h2, avs2, avd2, src, dst, z128, z16)
    out = _tc_combine_final(n2, d2, b2)
    return out[:N]
